# Initial kernel scaffold; baseline (speedup 1.0000x reference)
import jax
import jax.numpy as jnp
from jax.experimental import pallas as pl


def kernel(coords, features, W_mlp1, b_mlp1, W_lse1, b_lse1, g_lse1, be_lse1, W_score1, W_pool1, b_pool1, g_pool1, be_pool1, W_lse2, b_lse2, g_lse2, be_lse2, W_score2, W_pool2, b_pool2, g_pool2, be_pool2, W_mlp2, b_mlp2, W_sc, b_sc, g_sc, be_sc):
    raise NotImplementedError("write your pallas kernel here")


# speedup vs baseline: 8.2884x; 8.2884x over previous
import functools
import jax
import jax.numpy as jnp
from jax import lax
from jax.experimental import pallas as pl
from jax.experimental.pallas import tpu as pltpu
from jax.experimental.pallas import tpu_sc as plsc

B = 2
N = 4096
K = 16
DIN = 32
DOUT = 64
D2 = DOUT // 2
P = B * N
PK = P * K
EPS = 1e-6

TR = 256
TP = 256
NW = 32
CHUNK = PK // NW

f32 = jnp.float32


def _knn_body(c2_ref, ct_ref, idx_ref, dist_ref):
    g = pl.program_id(0)
    b = g // (N // TR)
    ct = ct_ref[0]
    sq_c = jnp.sum(ct * ct, axis=0, keepdims=True)
    r = c2_ref[...]
    sq_r = jnp.sum(r * r, axis=1, keepdims=True)
    gmat = jnp.dot(r, ct, preferred_element_type=f32)
    d = sq_r + sq_c - 2.0 * gmat
    col = lax.broadcasted_iota(jnp.int32, (TR, N), 1)
    kcol = lax.broadcasted_iota(jnp.int32, (TR, K), 1)
    idx_acc = jnp.zeros((TR, K), jnp.int32)
    dist_acc = jnp.zeros((TR, K), f32)
    for k in range(K):
        m = jnp.min(d, axis=1, keepdims=True)
        sel = jnp.where(d == m, col, N)
        j = jnp.min(sel, axis=1, keepdims=True)
        d = jnp.where(col == j, jnp.inf, d)
        idx_acc = jnp.where(kcol == k, j + b * N, idx_acc)
        dist_acc = jnp.where(kcol == k, jnp.maximum(m, 0.0), dist_acc)
    idx_ref[...] = idx_acc
    dist_ref[...] = dist_acc


def _knn(c2, ct):
    return pl.pallas_call(
        _knn_body,
        grid=(P // TR,),
        in_specs=[
            pl.BlockSpec((TR, 3), lambda g: (g, 0)),
            pl.BlockSpec((1, 3, N), lambda g: (g // (N // TR), 0, 0)),
        ],
        out_specs=[
            pl.BlockSpec((TR, K), lambda g: (g, 0)),
            pl.BlockSpec((TR, K), lambda g: (g, 0)),
        ],
        out_shape=[
            jax.ShapeDtypeStruct((P, K), jnp.int32),
            jax.ShapeDtypeStruct((P, K), f32),
        ],
    )(c2, ct)


SUB = 512


def _gather_body(tab_hbm, gidx_hbm, out_hbm, idx_v, rows_v, sem):
    wid = lax.axis_index("s") * 2 + lax.axis_index("c")
    base = wid * CHUNK
    pltpu.sync_copy(gidx_hbm.at[pl.ds(base, CHUNK)], idx_v)

    @pl.loop(0, CHUNK // SUB)
    def _(s):
        off = s * SUB
        pltpu.async_copy(tab_hbm.at[idx_v.at[pl.ds(off, SUB)]],
                         rows_v, sem).wait()
        pltpu.sync_copy(rows_v, out_hbm.at[pl.ds(base + off, SUB)])


def _gather_rows(tab, gidx):
    run = functools.partial(
        pl.kernel,
        out_type=jax.ShapeDtypeStruct((PK, 128), f32),
        mesh=plsc.VectorSubcoreMesh(core_axis_name="c", subcore_axis_name="s"),
        scratch_types=[
            pltpu.VMEM((CHUNK,), jnp.int32),
            pltpu.VMEM((SUB, 128), f32),
            pltpu.SemaphoreType.DMA,
        ],
    )(_gather_body)
    return run(tab, gidx)


def _fpath_body(f_ref, wm1_ref, bm1_ref, wsc_ref, bsc_ref,
                x1_ref, ysc_ref, ssc_ref, acc_ref):
    g = pl.program_id(0)
    fv = f_ref[...]
    x1 = jnp.dot(fv, wm1_ref[...], preferred_element_type=f32) + bm1_ref[...]
    x1_ref[...] = jnp.where(x1 >= 0, x1, 0.2 * x1)
    ysc = jnp.dot(fv, wsc_ref[...], preferred_element_type=f32) + bsc_ref[...]
    ysc_ref[...] = ysc

    @pl.when(g == 0)
    def _():
        acc_ref[...] = jnp.zeros_like(acc_ref)

    s = jnp.sum(ysc, axis=0, keepdims=True)
    ss = jnp.sum(ysc * ysc, axis=0, keepdims=True)
    acc_ref[...] += jnp.concatenate([s, ss], axis=0)

    @pl.when(g == pl.num_programs(0) - 1)
    def _():
        ssc_ref[...] = acc_ref[...]


def _fpath(fmat, wm1t, bm1, wsct, bsc):
    return pl.pallas_call(
        _fpath_body,
        grid=(P // TP,),
        in_specs=[
            pl.BlockSpec((TP, DIN), lambda g: (g, 0)),
            pl.BlockSpec((DIN, DIN), lambda g: (0, 0)),
            pl.BlockSpec((1, DIN), lambda g: (0, 0)),
            pl.BlockSpec((DIN, 2 * DOUT), lambda g: (0, 0)),
            pl.BlockSpec((1, 2 * DOUT), lambda g: (0, 0)),
        ],
        out_specs=[
            pl.BlockSpec((TP, DIN), lambda g: (g, 0)),
            pl.BlockSpec((TP, 2 * DOUT), lambda g: (g, 0)),
            pl.BlockSpec((2, 2 * DOUT), lambda g: (0, 0)),
        ],
        out_shape=[
            jax.ShapeDtypeStruct((P, DIN), f32),
            jax.ShapeDtypeStruct((P, 2 * DOUT), f32),
            jax.ShapeDtypeStruct((2, 2 * DOUT), f32),
        ],
        scratch_shapes=[pltpu.VMEM((2, 2 * DOUT), f32)],
    )(fmat, wm1t, bm1, wsct, bsc)


def _y_terms(c, nbr16, dist, wxt, wnt, wd, bv):
    cw = jnp.dot(c, wxt, preferred_element_type=f32)
    nb = nbr16[:, 0:3]
    nw_ = jnp.dot(nb, wnt, preferred_element_type=f32)
    y = (cw[:, None, :] + nw_.reshape(TP, K, D2)
         + dist[:, :, None] * wd.reshape(1, 1, D2) + bv.reshape(1, 1, D2))
    return y


def _encstats_body(c_ref, nbr_ref, dist_ref,
                   wxt1_ref, wnt1_ref, wd1_ref, b1_ref,
                   wxt2_ref, wnt2_ref, wd2_ref, b2_ref,
                   s1_ref, s2_ref, acc1_ref, acc2_ref):
    g = pl.program_id(0)
    c = c_ref[...]
    nbr16 = nbr_ref[...]
    dist = dist_ref[...]

    @pl.when(g == 0)
    def _():
        acc1_ref[...] = jnp.zeros_like(acc1_ref)
        acc2_ref[...] = jnp.zeros_like(acc2_ref)

    for (wxt, wnt, wd, bv, acc) in (
            (wxt1_ref, wnt1_ref, wd1_ref, b1_ref, acc1_ref),
            (wxt2_ref, wnt2_ref, wd2_ref, b2_ref, acc2_ref)):
        y = _y_terms(c, nbr16, dist, wxt[...], wnt[...], wd[...], bv[...])
        yf = y.reshape(TP * K, D2)
        s = jnp.sum(yf, axis=0, keepdims=True)
        ss = jnp.sum(yf * yf, axis=0, keepdims=True)
        acc[...] += jnp.concatenate([s, ss], axis=0)

    @pl.when(g == pl.num_programs(0) - 1)
    def _():
        s1_ref[...] = acc1_ref[...]
        s2_ref[...] = acc2_ref[...]


def _encstats(c2, nbr, dist, wxt1, wnt1, wd1, b1, wxt2, wnt2, wd2, b2):
    wspec = [
        pl.BlockSpec((3, D2), lambda g: (0, 0)),
        pl.BlockSpec((3, D2), lambda g: (0, 0)),
        pl.BlockSpec((1, D2), lambda g: (0, 0)),
        pl.BlockSpec((1, D2), lambda g: (0, 0)),
    ]
    return pl.pallas_call(
        _encstats_body,
        grid=(P // TP,),
        in_specs=[
            pl.BlockSpec((TP, 3), lambda g: (g, 0)),
            pl.BlockSpec((TP * K, 128), lambda g: (g, 0)),
            pl.BlockSpec((TP, K), lambda g: (g, 0)),
        ] + wspec + wspec,
        out_specs=[
            pl.BlockSpec((2, D2), lambda g: (0, 0)),
            pl.BlockSpec((2, D2), lambda g: (0, 0)),
        ],
        out_shape=[
            jax.ShapeDtypeStruct((2, D2), f32),
            jax.ShapeDtypeStruct((2, D2), f32),
        ],
        scratch_shapes=[pltpu.VMEM((2, D2), f32), pltpu.VMEM((2, D2), f32)],
    )(c2, nbr, dist, wxt1, wnt1, wd1, b1, wxt2, wnt2, wd2, b2)


def _bn_coeffs(stats, gv, bev, cnt):
    m = stats[0:1, :] / cnt
    v = stats[1:2, :] / cnt - m * m
    scale = gv / jnp.sqrt(v + EPS)
    shift = bev - m * scale
    return scale, shift


def _attpool(xb, wst, K_, TP_, C):
    s = jnp.dot(xb.reshape(TP_ * K_, C), wst,
                preferred_element_type=f32).reshape(TP_, K_, C)
    mx = s[:, 0, :]
    for k in range(1, K_):
        mx = jnp.maximum(mx, s[:, k, :])
    e = jnp.exp(s - mx[:, None, :])
    den = e[:, 0, :]
    for k in range(1, K_):
        den = den + e[:, k, :]
    pooled = (e[:, 0, :] / den) * xb[:, 0, :]
    for k in range(1, K_):
        pooled = pooled + (e[:, k, :] / den) * xb[:, k, :]
    return pooled


def _stage1_body(c_ref, nbr_ref, dist_ref, x1_ref, st1_ref,
                 wxt_ref, wnt_ref, wd_ref, b_ref, g1_ref, be1_ref,
                 wst_ref, wpt_ref, bp_ref,
                 z1_ref, sz_ref, acc_ref):
    g = pl.program_id(0)
    scale, shift = _bn_coeffs(st1_ref[...], g1_ref[...], be1_ref[...],
                              float(PK))
    y = _y_terms(c_ref[...], nbr_ref[...], dist_ref[...],
                 wxt_ref[...], wnt_ref[...], wd_ref[...], b_ref[...])
    enc = y * scale.reshape(1, 1, D2) + shift.reshape(1, 1, D2)
    enc = jnp.maximum(enc, 0.0)
    x1b = jnp.broadcast_to(x1_ref[...][:, None, :], (TP, K, D2))
    xb = jnp.concatenate([enc, x1b], axis=2)
    pooled = _attpool(xb, wst_ref[...], K, TP, DOUT)
    z = jnp.dot(pooled, wpt_ref[...], preferred_element_type=f32) + bp_ref[...]
    z1_ref[...] = z

    @pl.when(g == 0)
    def _():
        acc_ref[...] = jnp.zeros_like(acc_ref)

    s = jnp.sum(z, axis=0, keepdims=True)
    ss = jnp.sum(z * z, axis=0, keepdims=True)
    acc_ref[...] += jnp.concatenate([s, ss], axis=0)

    @pl.when(g == pl.num_programs(0) - 1)
    def _():
        sz_ref[...] = acc_ref[...]


def _stage1(c2, nbr, dist, x1, st1, wxt, wnt, wd, bv, g1, be1, wst, wpt, bp):
    return pl.pallas_call(
        _stage1_body,
        grid=(P // TP,),
        in_specs=[
            pl.BlockSpec((TP, 3), lambda g: (g, 0)),
            pl.BlockSpec((TP * K, 128), lambda g: (g, 0)),
            pl.BlockSpec((TP, K), lambda g: (g, 0)),
            pl.BlockSpec((TP, D2), lambda g: (g, 0)),
            pl.BlockSpec((2, D2), lambda g: (0, 0)),
            pl.BlockSpec((3, D2), lambda g: (0, 0)),
            pl.BlockSpec((3, D2), lambda g: (0, 0)),
            pl.BlockSpec((1, D2), lambda g: (0, 0)),
            pl.BlockSpec((1, D2), lambda g: (0, 0)),
            pl.BlockSpec((1, D2), lambda g: (0, 0)),
            pl.BlockSpec((1, D2), lambda g: (0, 0)),
            pl.BlockSpec((DOUT, DOUT), lambda g: (0, 0)),
            pl.BlockSpec((DOUT, D2), lambda g: (0, 0)),
            pl.BlockSpec((1, D2), lambda g: (0, 0)),
        ],
        out_specs=[
            pl.BlockSpec((TP, D2), lambda g: (g, 0)),
            pl.BlockSpec((2, D2), lambda g: (0, 0)),
        ],
        out_shape=[
            jax.ShapeDtypeStruct((P, D2), f32),
            jax.ShapeDtypeStruct((2, D2), f32),
        ],
        scratch_shapes=[pltpu.VMEM((2, D2), f32)],
    )(c2, nbr, dist, x1, st1, wxt, wnt, wd, bv, g1, be1, wst, wpt, bp)


def _stage2_body(c_ref, nbr_ref, dist_ref, z1_ref, sz1_ref, st2_ref,
                 gp1_ref, bep1_ref,
                 wxt_ref, wnt_ref, wd_ref, b_ref, g2_ref, be2_ref,
                 wst_ref, wpt_ref, bp_ref,
                 z2_ref, sz2_ref, acc_ref):
    g = pl.program_id(0)
    zscale, zshift = _bn_coeffs(sz1_ref[...], gp1_ref[...], bep1_ref[...],
                                float(P))
    x2 = jnp.maximum(z1_ref[...] * zscale + zshift, 0.0)
    escale, eshift = _bn_coeffs(st2_ref[...], g2_ref[...], be2_ref[...],
                                float(PK))
    y = _y_terms(c_ref[...], nbr_ref[...], dist_ref[...],
                 wxt_ref[...], wnt_ref[...], wd_ref[...], b_ref[...])
    enc = jnp.maximum(y * escale.reshape(1, 1, D2)
                      + eshift.reshape(1, 1, D2), 0.0)
    x2b = jnp.broadcast_to(x2[:, None, :], (TP, K, D2))
    xb = jnp.concatenate([enc, x2b], axis=2)
    pooled = _attpool(xb, wst_ref[...], K, TP, DOUT)
    z = jnp.dot(pooled, wpt_ref[...], preferred_element_type=f32) + bp_ref[...]
    z2_ref[...] = z

    @pl.when(g == 0)
    def _():
        acc_ref[...] = jnp.zeros_like(acc_ref)

    s = jnp.sum(z, axis=0, keepdims=True)
    ss = jnp.sum(z * z, axis=0, keepdims=True)
    acc_ref[...] += jnp.concatenate([s, ss], axis=0)

    @pl.when(g == pl.num_programs(0) - 1)
    def _():
        sz2_ref[...] = acc_ref[...]


def _stage2(c2, nbr, dist, z1, sz1, st2, gp1, bep1,
            wxt, wnt, wd, bv, g2, be2, wst, wpt, bp):
    return pl.pallas_call(
        _stage2_body,
        grid=(P // TP,),
        in_specs=[
            pl.BlockSpec((TP, 3), lambda g: (g, 0)),
            pl.BlockSpec((TP * K, 128), lambda g: (g, 0)),
            pl.BlockSpec((TP, K), lambda g: (g, 0)),
            pl.BlockSpec((TP, D2), lambda g: (g, 0)),
            pl.BlockSpec((2, D2), lambda g: (0, 0)),
            pl.BlockSpec((2, D2), lambda g: (0, 0)),
            pl.BlockSpec((1, D2), lambda g: (0, 0)),
            pl.BlockSpec((1, D2), lambda g: (0, 0)),
            pl.BlockSpec((3, D2), lambda g: (0, 0)),
            pl.BlockSpec((3, D2), lambda g: (0, 0)),
            pl.BlockSpec((1, D2), lambda g: (0, 0)),
            pl.BlockSpec((1, D2), lambda g: (0, 0)),
            pl.BlockSpec((1, D2), lambda g: (0, 0)),
            pl.BlockSpec((1, D2), lambda g: (0, 0)),
            pl.BlockSpec((DOUT, DOUT), lambda g: (0, 0)),
            pl.BlockSpec((DOUT, DOUT), lambda g: (0, 0)),
            pl.BlockSpec((1, DOUT), lambda g: (0, 0)),
        ],
        out_specs=[
            pl.BlockSpec((TP, DOUT), lambda g: (g, 0)),
            pl.BlockSpec((2, DOUT), lambda g: (0, 0)),
        ],
        out_shape=[
            jax.ShapeDtypeStruct((P, DOUT), f32),
            jax.ShapeDtypeStruct((2, DOUT), f32),
        ],
        scratch_shapes=[pltpu.VMEM((2, DOUT), f32)],
    )(c2, nbr, dist, z1, sz1, st2, gp1, bep1,
      wxt, wnt, wd, bv, g2, be2, wst, wpt, bp)


def _final_body(z2_ref, sz2_ref, gp2_ref, bep2_ref,
                ysc_ref, ssc_ref, gsc_ref, besc_ref,
                wm2_ref, bm2_ref, out_ref):
    zscale, zshift = _bn_coeffs(sz2_ref[...], gp2_ref[...], bep2_ref[...],
                                float(P))
    x3 = jnp.maximum(z2_ref[...] * zscale + zshift, 0.0)
    sscale, sshift = _bn_coeffs(ssc_ref[...], gsc_ref[...], besc_ref[...],
                                float(P))
    sc = ysc_ref[...] * sscale + sshift
    out = jnp.dot(x3, wm2_ref[...], preferred_element_type=f32) \
        + bm2_ref[...] + sc
    out_ref[...] = jnp.where(out >= 0, out, 0.01 * out)


def _final(z2, sz2, gp2, bep2, ysc, ssc, gsc, besc, wm2t, bm2):
    return pl.pallas_call(
        _final_body,
        grid=(P // TP,),
        in_specs=[
            pl.BlockSpec((TP, DOUT), lambda g: (g, 0)),
            pl.BlockSpec((2, DOUT), lambda g: (0, 0)),
            pl.BlockSpec((1, DOUT), lambda g: (0, 0)),
            pl.BlockSpec((1, DOUT), lambda g: (0, 0)),
            pl.BlockSpec((TP, 2 * DOUT), lambda g: (g, 0)),
            pl.BlockSpec((2, 2 * DOUT), lambda g: (0, 0)),
            pl.BlockSpec((1, 2 * DOUT), lambda g: (0, 0)),
            pl.BlockSpec((1, 2 * DOUT), lambda g: (0, 0)),
            pl.BlockSpec((DOUT, 2 * DOUT), lambda g: (0, 0)),
            pl.BlockSpec((1, 2 * DOUT), lambda g: (0, 0)),
        ],
        out_specs=pl.BlockSpec((TP, 2 * DOUT), lambda g: (g, 0)),
        out_shape=jax.ShapeDtypeStruct((P, 2 * DOUT), f32),
    )(z2, sz2, gp2, bep2, ysc, ssc, gsc, besc, wm2t, bm2)


def _split_lse(w):
    wx = (w[:, 0:3] + w[:, 6:9]).T
    wn = (w[:, 3:6] - w[:, 6:9]).T
    wd = w[:, 9].reshape(1, D2)
    return wx, wn, wd


def kernel(coords, features, W_mlp1, b_mlp1, W_lse1, b_lse1, g_lse1, be_lse1,
           W_score1, W_pool1, b_pool1, g_pool1, be_pool1, W_lse2, b_lse2,
           g_lse2, be_lse2, W_score2, W_pool2, b_pool2, g_pool2, be_pool2,
           W_mlp2, b_mlp2, W_sc, b_sc, g_sc, be_sc):
    c2 = coords.reshape(P, 3)
    ct = coords.transpose(0, 2, 1)
    fmat = features.reshape(B, DIN, N).transpose(0, 2, 1).reshape(P, DIN)
    tab = jnp.pad(c2, ((0, 0), (0, 125)))

    idx, dist = _knn(c2, ct)
    nbr = _gather_rows(tab, idx.reshape(PK))

    x1, ysc, ssc = _fpath(fmat, W_mlp1.T, b_mlp1.reshape(1, DIN),
                          W_sc.T, b_sc.reshape(1, 2 * DOUT))

    wx1, wn1, wd1 = _split_lse(W_lse1)
    wx2, wn2, wd2 = _split_lse(W_lse2)
    b1 = b_lse1.reshape(1, D2)
    b2 = b_lse2.reshape(1, D2)

    st1, st2 = _encstats(c2, nbr, dist, wx1, wn1, wd1, b1, wx2, wn2, wd2, b2)

    z1, sz1 = _stage1(c2, nbr, dist, x1, st1, wx1, wn1, wd1, b1,
                      g_lse1.reshape(1, D2), be_lse1.reshape(1, D2),
                      W_score1.T, W_pool1.T, b_pool1.reshape(1, D2))

    z2, sz2 = _stage2(c2, nbr, dist, z1, sz1, st2,
                      g_pool1.reshape(1, D2), be_pool1.reshape(1, D2),
                      wx2, wn2, wd2, b2,
                      g_lse2.reshape(1, D2), be_lse2.reshape(1, D2),
                      W_score2.T, W_pool2.T, b_pool2.reshape(1, DOUT))

    out = _final(z2, sz2, g_pool2.reshape(1, DOUT), be_pool2.reshape(1, DOUT),
                 ysc, ssc, g_sc.reshape(1, 2 * DOUT),
                 be_sc.reshape(1, 2 * DOUT),
                 W_mlp2.T, b_mlp2.reshape(1, 2 * DOUT))

    return out.reshape(B, N, 2 * DOUT).transpose(0, 2, 1)[:, :, :, None]


# speedup vs baseline: 13.3244x; 1.6076x over previous
import functools
import jax
import jax.numpy as jnp
from jax import lax
from jax.experimental import pallas as pl
from jax.experimental.pallas import tpu as pltpu
from jax.experimental.pallas import tpu_sc as plsc

B = 2
N = 4096
K = 16
DIN = 32
DOUT = 64
D2 = DOUT // 2
P = B * N
PK = P * K
EPS = 1e-6

TR = 256
TP = 256
NW = 32
CHUNK = PK // NW

f32 = jnp.float32


def _knn_body(c2_ref, ct_ref, idx_ref, dist_ref):
    g = pl.program_id(0)
    b = g // (N // TR)
    ct = ct_ref[0]
    sq_c = jnp.sum(ct * ct, axis=0, keepdims=True)
    r = c2_ref[...]
    sq_r = jnp.sum(r * r, axis=1, keepdims=True)
    gmat = jnp.dot(r, ct, preferred_element_type=f32)
    d = sq_r + sq_c - 2.0 * gmat
    col = lax.broadcasted_iota(jnp.int32, (TR, N), 1)
    kcol = lax.broadcasted_iota(jnp.int32, (TR, K), 1)
    packed = (lax.bitcast_convert_type(d, jnp.int32) & ~jnp.int32(0xFFF)) | col
    dead = jnp.int32(0x7FFFFFFF)
    idx_acc = jnp.zeros((TR, K), jnp.int32)
    dist_acc = jnp.zeros((TR, K), f32)
    for k in range(K):
        mp = jnp.min(packed, axis=1, keepdims=True)
        packed = jnp.where(packed == mp, dead, packed)
        dv = lax.bitcast_convert_type(mp & ~jnp.int32(0xFFF), f32)
        idx_acc = jnp.where(kcol == k, (mp & jnp.int32(0xFFF)) + b * N, idx_acc)
        dist_acc = jnp.where(kcol == k, jnp.maximum(dv, 0.0), dist_acc)
    idx_ref[...] = idx_acc
    dist_ref[...] = dist_acc


def _knn(c2, ct):
    return pl.pallas_call(
        _knn_body,
        grid=(P // TR,),
        in_specs=[
            pl.BlockSpec((TR, 3), lambda g: (g, 0)),
            pl.BlockSpec((1, 3, N), lambda g: (g // (N // TR), 0, 0)),
        ],
        out_specs=[
            pl.BlockSpec((TR, K), lambda g: (g, 0)),
            pl.BlockSpec((TR, K), lambda g: (g, 0)),
        ],
        out_shape=[
            jax.ShapeDtypeStruct((P, K), jnp.int32),
            jax.ShapeDtypeStruct((P, K), f32),
        ],
    )(c2, ct)


SUB = 512


def _gather_body(tab_hbm, gidx_hbm, out_hbm, idx_v, rows_v, sem):
    wid = lax.axis_index("s") * 2 + lax.axis_index("c")
    base = wid * CHUNK
    pltpu.sync_copy(gidx_hbm.at[pl.ds(base, CHUNK)], idx_v)

    @pl.loop(0, CHUNK // SUB)
    def _(s):
        off = s * SUB
        pltpu.async_copy(tab_hbm.at[idx_v.at[pl.ds(off, SUB)]],
                         rows_v, sem).wait()
        pltpu.sync_copy(rows_v, out_hbm.at[pl.ds(base + off, SUB)])


def _gather_rows(tab, gidx):
    run = functools.partial(
        pl.kernel,
        out_type=jax.ShapeDtypeStruct((PK, 128), f32),
        mesh=plsc.VectorSubcoreMesh(core_axis_name="c", subcore_axis_name="s"),
        scratch_types=[
            pltpu.VMEM((CHUNK,), jnp.int32),
            pltpu.VMEM((SUB, 128), f32),
            pltpu.SemaphoreType.DMA,
        ],
    )(_gather_body)
    return run(tab, gidx)


def _fpath_body(f_ref, wm1_ref, bm1_ref, wsc_ref, bsc_ref,
                x1_ref, ysc_ref, ssc_ref, acc_ref):
    g = pl.program_id(0)
    fv = f_ref[...]
    x1 = jnp.dot(fv, wm1_ref[...], preferred_element_type=f32) + bm1_ref[...]
    x1_ref[...] = jnp.where(x1 >= 0, x1, 0.2 * x1)
    ysc = jnp.dot(fv, wsc_ref[...], preferred_element_type=f32) + bsc_ref[...]
    ysc_ref[...] = ysc

    @pl.when(g == 0)
    def _():
        acc_ref[...] = jnp.zeros_like(acc_ref)

    s = jnp.sum(ysc, axis=0, keepdims=True)
    ss = jnp.sum(ysc * ysc, axis=0, keepdims=True)
    acc_ref[...] += jnp.concatenate([s, ss], axis=0)

    @pl.when(g == pl.num_programs(0) - 1)
    def _():
        ssc_ref[...] = acc_ref[...]


def _fpath(fmat, wm1t, bm1, wsct, bsc):
    return pl.pallas_call(
        _fpath_body,
        grid=(P // TP,),
        in_specs=[
            pl.BlockSpec((TP, DIN), lambda g: (g, 0)),
            pl.BlockSpec((DIN, DIN), lambda g: (0, 0)),
            pl.BlockSpec((1, DIN), lambda g: (0, 0)),
            pl.BlockSpec((DIN, 2 * DOUT), lambda g: (0, 0)),
            pl.BlockSpec((1, 2 * DOUT), lambda g: (0, 0)),
        ],
        out_specs=[
            pl.BlockSpec((TP, DIN), lambda g: (g, 0)),
            pl.BlockSpec((TP, 2 * DOUT), lambda g: (g, 0)),
            pl.BlockSpec((2, 2 * DOUT), lambda g: (0, 0)),
        ],
        out_shape=[
            jax.ShapeDtypeStruct((P, DIN), f32),
            jax.ShapeDtypeStruct((P, 2 * DOUT), f32),
            jax.ShapeDtypeStruct((2, 2 * DOUT), f32),
        ],
        scratch_shapes=[pltpu.VMEM((2, 2 * DOUT), f32)],
    )(fmat, wm1t, bm1, wsct, bsc)


def _y_terms(c, nbr16, dist, wxt, wnt, wd, bv):
    cw = jnp.dot(c, wxt, preferred_element_type=f32)
    nb = nbr16[:, 0:3]
    nw_ = jnp.dot(nb, wnt, preferred_element_type=f32)
    y = (cw[:, None, :] + nw_.reshape(TP, K, D2)
         + dist[:, :, None] * wd.reshape(1, 1, D2) + bv.reshape(1, 1, D2))
    return y


def _encstats_body(c_ref, nbr_ref, dist_ref,
                   wxt1_ref, wnt1_ref, wd1_ref, b1_ref,
                   wxt2_ref, wnt2_ref, wd2_ref, b2_ref,
                   s1_ref, s2_ref, acc1_ref, acc2_ref):
    g = pl.program_id(0)
    c = c_ref[...]
    nbr16 = nbr_ref[...]
    dist = dist_ref[...]

    @pl.when(g == 0)
    def _():
        acc1_ref[...] = jnp.zeros_like(acc1_ref)
        acc2_ref[...] = jnp.zeros_like(acc2_ref)

    for (wxt, wnt, wd, bv, acc) in (
            (wxt1_ref, wnt1_ref, wd1_ref, b1_ref, acc1_ref),
            (wxt2_ref, wnt2_ref, wd2_ref, b2_ref, acc2_ref)):
        y = _y_terms(c, nbr16, dist, wxt[...], wnt[...], wd[...], bv[...])
        yf = y.reshape(TP * K, D2)
        s = jnp.sum(yf, axis=0, keepdims=True)
        ss = jnp.sum(yf * yf, axis=0, keepdims=True)
        acc[...] += jnp.concatenate([s, ss], axis=0)

    @pl.when(g == pl.num_programs(0) - 1)
    def _():
        s1_ref[...] = acc1_ref[...]
        s2_ref[...] = acc2_ref[...]


def _encstats(c2, nbr, dist, wxt1, wnt1, wd1, b1, wxt2, wnt2, wd2, b2):
    wspec = [
        pl.BlockSpec((3, D2), lambda g: (0, 0)),
        pl.BlockSpec((3, D2), lambda g: (0, 0)),
        pl.BlockSpec((1, D2), lambda g: (0, 0)),
        pl.BlockSpec((1, D2), lambda g: (0, 0)),
    ]
    return pl.pallas_call(
        _encstats_body,
        grid=(P // TP,),
        in_specs=[
            pl.BlockSpec((TP, 3), lambda g: (g, 0)),
            pl.BlockSpec((TP * K, 128), lambda g: (g, 0)),
            pl.BlockSpec((TP, K), lambda g: (g, 0)),
        ] + wspec + wspec,
        out_specs=[
            pl.BlockSpec((2, D2), lambda g: (0, 0)),
            pl.BlockSpec((2, D2), lambda g: (0, 0)),
        ],
        out_shape=[
            jax.ShapeDtypeStruct((2, D2), f32),
            jax.ShapeDtypeStruct((2, D2), f32),
        ],
        scratch_shapes=[pltpu.VMEM((2, D2), f32), pltpu.VMEM((2, D2), f32)],
    )(c2, nbr, dist, wxt1, wnt1, wd1, b1, wxt2, wnt2, wd2, b2)


def _bn_coeffs(stats, gv, bev, cnt):
    m = stats[0:1, :] / cnt
    v = stats[1:2, :] / cnt - m * m
    scale = gv / jnp.sqrt(v + EPS)
    shift = bev - m * scale
    return scale, shift


def _attpool(xb, wst, K_, TP_, C):
    s = jnp.dot(xb.reshape(TP_ * K_, C), wst,
                preferred_element_type=f32).reshape(TP_, K_, C)
    mx = jnp.max(s, axis=1, keepdims=True)
    e = jnp.exp(s - mx)
    rden = 1.0 / jnp.sum(e, axis=1, keepdims=True)
    return jnp.sum((e * rden) * xb, axis=1)


def _stage1_body(c_ref, nbr_ref, dist_ref, x1_ref, st1_ref,
                 wxt_ref, wnt_ref, wd_ref, b_ref, g1_ref, be1_ref,
                 wst_ref, wpt_ref, bp_ref,
                 z1_ref, sz_ref, acc_ref):
    g = pl.program_id(0)
    scale, shift = _bn_coeffs(st1_ref[...], g1_ref[...], be1_ref[...],
                              float(PK))
    y = _y_terms(c_ref[...], nbr_ref[...], dist_ref[...],
                 wxt_ref[...], wnt_ref[...], wd_ref[...], b_ref[...])
    enc = y * scale.reshape(1, 1, D2) + shift.reshape(1, 1, D2)
    enc = jnp.maximum(enc, 0.0)
    x1b = jnp.broadcast_to(x1_ref[...][:, None, :], (TP, K, D2))
    xb = jnp.concatenate([enc, x1b], axis=2)
    pooled = _attpool(xb, wst_ref[...], K, TP, DOUT)
    z = jnp.dot(pooled, wpt_ref[...], preferred_element_type=f32) + bp_ref[...]
    z1_ref[...] = z

    @pl.when(g == 0)
    def _():
        acc_ref[...] = jnp.zeros_like(acc_ref)

    s = jnp.sum(z, axis=0, keepdims=True)
    ss = jnp.sum(z * z, axis=0, keepdims=True)
    acc_ref[...] += jnp.concatenate([s, ss], axis=0)

    @pl.when(g == pl.num_programs(0) - 1)
    def _():
        sz_ref[...] = acc_ref[...]


def _stage1(c2, nbr, dist, x1, st1, wxt, wnt, wd, bv, g1, be1, wst, wpt, bp):
    return pl.pallas_call(
        _stage1_body,
        grid=(P // TP,),
        in_specs=[
            pl.BlockSpec((TP, 3), lambda g: (g, 0)),
            pl.BlockSpec((TP * K, 128), lambda g: (g, 0)),
            pl.BlockSpec((TP, K), lambda g: (g, 0)),
            pl.BlockSpec((TP, D2), lambda g: (g, 0)),
            pl.BlockSpec((2, D2), lambda g: (0, 0)),
            pl.BlockSpec((3, D2), lambda g: (0, 0)),
            pl.BlockSpec((3, D2), lambda g: (0, 0)),
            pl.BlockSpec((1, D2), lambda g: (0, 0)),
            pl.BlockSpec((1, D2), lambda g: (0, 0)),
            pl.BlockSpec((1, D2), lambda g: (0, 0)),
            pl.BlockSpec((1, D2), lambda g: (0, 0)),
            pl.BlockSpec((DOUT, DOUT), lambda g: (0, 0)),
            pl.BlockSpec((DOUT, D2), lambda g: (0, 0)),
            pl.BlockSpec((1, D2), lambda g: (0, 0)),
        ],
        out_specs=[
            pl.BlockSpec((TP, D2), lambda g: (g, 0)),
            pl.BlockSpec((2, D2), lambda g: (0, 0)),
        ],
        out_shape=[
            jax.ShapeDtypeStruct((P, D2), f32),
            jax.ShapeDtypeStruct((2, D2), f32),
        ],
        scratch_shapes=[pltpu.VMEM((2, D2), f32)],
    )(c2, nbr, dist, x1, st1, wxt, wnt, wd, bv, g1, be1, wst, wpt, bp)


def _stage2_body(c_ref, nbr_ref, dist_ref, z1_ref, sz1_ref, st2_ref,
                 gp1_ref, bep1_ref,
                 wxt_ref, wnt_ref, wd_ref, b_ref, g2_ref, be2_ref,
                 wst_ref, wpt_ref, bp_ref,
                 z2_ref, sz2_ref, acc_ref):
    g = pl.program_id(0)
    zscale, zshift = _bn_coeffs(sz1_ref[...], gp1_ref[...], bep1_ref[...],
                                float(P))
    x2 = jnp.maximum(z1_ref[...] * zscale + zshift, 0.0)
    escale, eshift = _bn_coeffs(st2_ref[...], g2_ref[...], be2_ref[...],
                                float(PK))
    y = _y_terms(c_ref[...], nbr_ref[...], dist_ref[...],
                 wxt_ref[...], wnt_ref[...], wd_ref[...], b_ref[...])
    enc = jnp.maximum(y * escale.reshape(1, 1, D2)
                      + eshift.reshape(1, 1, D2), 0.0)
    x2b = jnp.broadcast_to(x2[:, None, :], (TP, K, D2))
    xb = jnp.concatenate([enc, x2b], axis=2)
    pooled = _attpool(xb, wst_ref[...], K, TP, DOUT)
    z = jnp.dot(pooled, wpt_ref[...], preferred_element_type=f32) + bp_ref[...]
    z2_ref[...] = z

    @pl.when(g == 0)
    def _():
        acc_ref[...] = jnp.zeros_like(acc_ref)

    s = jnp.sum(z, axis=0, keepdims=True)
    ss = jnp.sum(z * z, axis=0, keepdims=True)
    acc_ref[...] += jnp.concatenate([s, ss], axis=0)

    @pl.when(g == pl.num_programs(0) - 1)
    def _():
        sz2_ref[...] = acc_ref[...]


def _stage2(c2, nbr, dist, z1, sz1, st2, gp1, bep1,
            wxt, wnt, wd, bv, g2, be2, wst, wpt, bp):
    return pl.pallas_call(
        _stage2_body,
        grid=(P // TP,),
        in_specs=[
            pl.BlockSpec((TP, 3), lambda g: (g, 0)),
            pl.BlockSpec((TP * K, 128), lambda g: (g, 0)),
            pl.BlockSpec((TP, K), lambda g: (g, 0)),
            pl.BlockSpec((TP, D2), lambda g: (g, 0)),
            pl.BlockSpec((2, D2), lambda g: (0, 0)),
            pl.BlockSpec((2, D2), lambda g: (0, 0)),
            pl.BlockSpec((1, D2), lambda g: (0, 0)),
            pl.BlockSpec((1, D2), lambda g: (0, 0)),
            pl.BlockSpec((3, D2), lambda g: (0, 0)),
            pl.BlockSpec((3, D2), lambda g: (0, 0)),
            pl.BlockSpec((1, D2), lambda g: (0, 0)),
            pl.BlockSpec((1, D2), lambda g: (0, 0)),
            pl.BlockSpec((1, D2), lambda g: (0, 0)),
            pl.BlockSpec((1, D2), lambda g: (0, 0)),
            pl.BlockSpec((DOUT, DOUT), lambda g: (0, 0)),
            pl.BlockSpec((DOUT, DOUT), lambda g: (0, 0)),
            pl.BlockSpec((1, DOUT), lambda g: (0, 0)),
        ],
        out_specs=[
            pl.BlockSpec((TP, DOUT), lambda g: (g, 0)),
            pl.BlockSpec((2, DOUT), lambda g: (0, 0)),
        ],
        out_shape=[
            jax.ShapeDtypeStruct((P, DOUT), f32),
            jax.ShapeDtypeStruct((2, DOUT), f32),
        ],
        scratch_shapes=[pltpu.VMEM((2, DOUT), f32)],
    )(c2, nbr, dist, z1, sz1, st2, gp1, bep1,
      wxt, wnt, wd, bv, g2, be2, wst, wpt, bp)


def _final_body(z2_ref, sz2_ref, gp2_ref, bep2_ref,
                ysc_ref, ssc_ref, gsc_ref, besc_ref,
                wm2_ref, bm2_ref, out_ref):
    zscale, zshift = _bn_coeffs(sz2_ref[...], gp2_ref[...], bep2_ref[...],
                                float(P))
    x3 = jnp.maximum(z2_ref[...] * zscale + zshift, 0.0)
    sscale, sshift = _bn_coeffs(ssc_ref[...], gsc_ref[...], besc_ref[...],
                                float(P))
    sc = ysc_ref[...] * sscale + sshift
    out = jnp.dot(x3, wm2_ref[...], preferred_element_type=f32) \
        + bm2_ref[...] + sc
    out_ref[...] = jnp.where(out >= 0, out, 0.01 * out)


def _final(z2, sz2, gp2, bep2, ysc, ssc, gsc, besc, wm2t, bm2):
    return pl.pallas_call(
        _final_body,
        grid=(P // TP,),
        in_specs=[
            pl.BlockSpec((TP, DOUT), lambda g: (g, 0)),
            pl.BlockSpec((2, DOUT), lambda g: (0, 0)),
            pl.BlockSpec((1, DOUT), lambda g: (0, 0)),
            pl.BlockSpec((1, DOUT), lambda g: (0, 0)),
            pl.BlockSpec((TP, 2 * DOUT), lambda g: (g, 0)),
            pl.BlockSpec((2, 2 * DOUT), lambda g: (0, 0)),
            pl.BlockSpec((1, 2 * DOUT), lambda g: (0, 0)),
            pl.BlockSpec((1, 2 * DOUT), lambda g: (0, 0)),
            pl.BlockSpec((DOUT, 2 * DOUT), lambda g: (0, 0)),
            pl.BlockSpec((1, 2 * DOUT), lambda g: (0, 0)),
        ],
        out_specs=pl.BlockSpec((TP, 2 * DOUT), lambda g: (g, 0)),
        out_shape=jax.ShapeDtypeStruct((P, 2 * DOUT), f32),
    )(z2, sz2, gp2, bep2, ysc, ssc, gsc, besc, wm2t, bm2)


def _split_lse(w):
    wx = (w[:, 0:3] + w[:, 6:9]).T
    wn = (w[:, 3:6] - w[:, 6:9]).T
    wd = w[:, 9].reshape(1, D2)
    return wx, wn, wd


def kernel(coords, features, W_mlp1, b_mlp1, W_lse1, b_lse1, g_lse1, be_lse1,
           W_score1, W_pool1, b_pool1, g_pool1, be_pool1, W_lse2, b_lse2,
           g_lse2, be_lse2, W_score2, W_pool2, b_pool2, g_pool2, be_pool2,
           W_mlp2, b_mlp2, W_sc, b_sc, g_sc, be_sc):
    c2 = coords.reshape(P, 3)
    ct = coords.transpose(0, 2, 1)
    fmat = features.reshape(B, DIN, N).transpose(0, 2, 1).reshape(P, DIN)
    tab = jnp.pad(c2, ((0, 0), (0, 125)))

    idx, dist = _knn(c2, ct)
    nbr = _gather_rows(tab, idx.reshape(PK))

    x1, ysc, ssc = _fpath(fmat, W_mlp1.T, b_mlp1.reshape(1, DIN),
                          W_sc.T, b_sc.reshape(1, 2 * DOUT))

    wx1, wn1, wd1 = _split_lse(W_lse1)
    wx2, wn2, wd2 = _split_lse(W_lse2)
    b1 = b_lse1.reshape(1, D2)
    b2 = b_lse2.reshape(1, D2)

    st1, st2 = _encstats(c2, nbr, dist, wx1, wn1, wd1, b1, wx2, wn2, wd2, b2)

    z1, sz1 = _stage1(c2, nbr, dist, x1, st1, wx1, wn1, wd1, b1,
                      g_lse1.reshape(1, D2), be_lse1.reshape(1, D2),
                      W_score1.T, W_pool1.T, b_pool1.reshape(1, D2))

    z2, sz2 = _stage2(c2, nbr, dist, z1, sz1, st2,
                      g_pool1.reshape(1, D2), be_pool1.reshape(1, D2),
                      wx2, wn2, wd2, b2,
                      g_lse2.reshape(1, D2), be_lse2.reshape(1, D2),
                      W_score2.T, W_pool2.T, b_pool2.reshape(1, DOUT))

    out = _final(z2, sz2, g_pool2.reshape(1, DOUT), be_pool2.reshape(1, DOUT),
                 ysc, ssc, g_sc.reshape(1, 2 * DOUT),
                 be_sc.reshape(1, 2 * DOUT),
                 W_mlp2.T, b_mlp2.reshape(1, 2 * DOUT))

    return out.reshape(B, N, 2 * DOUT).transpose(0, 2, 1)[:, :, :, None]


# speedup vs baseline: 13.5117x; 1.0141x over previous
import functools
import jax
import jax.numpy as jnp
from jax import lax
from jax.experimental import pallas as pl
from jax.experimental.pallas import tpu as pltpu
from jax.experimental.pallas import tpu_sc as plsc

B = 2
N = 4096
K = 16
DIN = 32
DOUT = 64
D2 = DOUT // 2
P = B * N
PK = P * K
EPS = 1e-6

TR = 256
TP = 256
NW = 32
CHUNK = PK // NW

f32 = jnp.float32


def _knn_body(c2_ref, ct_ref, idx_ref, dist_ref):
    g = pl.program_id(0)
    b = g // (N // TR)
    ct = ct_ref[0]
    sq_c = jnp.sum(ct * ct, axis=0, keepdims=True)
    r = c2_ref[...]
    sq_r = jnp.sum(r * r, axis=1, keepdims=True)
    gmat = jnp.dot(r, ct, preferred_element_type=f32)
    d = sq_r + sq_c - 2.0 * gmat
    col = lax.broadcasted_iota(jnp.int32, (TR, N), 1)
    kcol = lax.broadcasted_iota(jnp.int32, (TR, K), 1)
    packed = (lax.bitcast_convert_type(d, jnp.int32) & ~jnp.int32(0xFFF)) | col
    dead = jnp.int32(0x7FFFFFFF)
    idx_acc = jnp.zeros((TR, K), jnp.int32)
    dist_acc = jnp.zeros((TR, K), f32)
    m = jnp.min(packed, axis=1, keepdims=True)
    for k in range(K):
        dv = lax.bitcast_convert_type(m & ~jnp.int32(0xFFF), f32)
        idx_acc = jnp.where(kcol == k, (m & jnp.int32(0xFFF)) + b * N, idx_acc)
        dist_acc = jnp.where(kcol == k, jnp.maximum(dv, 0.0), dist_acc)
        if k < K - 1:
            m = jnp.min(jnp.where(packed > m, packed, dead),
                        axis=1, keepdims=True)
    idx_ref[...] = idx_acc
    dist_ref[...] = dist_acc


def _knn(c2, ct):
    return pl.pallas_call(
        _knn_body,
        grid=(P // TR,),
        in_specs=[
            pl.BlockSpec((TR, 3), lambda g: (g, 0)),
            pl.BlockSpec((1, 3, N), lambda g: (g // (N // TR), 0, 0)),
        ],
        out_specs=[
            pl.BlockSpec((TR, K), lambda g: (g, 0)),
            pl.BlockSpec((TR, K), lambda g: (g, 0)),
        ],
        out_shape=[
            jax.ShapeDtypeStruct((P, K), jnp.int32),
            jax.ShapeDtypeStruct((P, K), f32),
        ],
    )(c2, ct)


SUB = 512


def _gather_body(tab_hbm, gidx_hbm, out_hbm, idx_v, rows_v, sem):
    wid = lax.axis_index("s") * 2 + lax.axis_index("c")
    base = wid * CHUNK
    pltpu.sync_copy(gidx_hbm.at[pl.ds(base, CHUNK)], idx_v)

    @pl.loop(0, CHUNK // SUB)
    def _(s):
        off = s * SUB
        pltpu.async_copy(tab_hbm.at[idx_v.at[pl.ds(off, SUB)]],
                         rows_v, sem).wait()
        pltpu.sync_copy(rows_v, out_hbm.at[pl.ds(base + off, SUB)])


def _gather_rows(tab, gidx):
    run = functools.partial(
        pl.kernel,
        out_type=jax.ShapeDtypeStruct((PK, 128), f32),
        mesh=plsc.VectorSubcoreMesh(core_axis_name="c", subcore_axis_name="s"),
        scratch_types=[
            pltpu.VMEM((CHUNK,), jnp.int32),
            pltpu.VMEM((SUB, 128), f32),
            pltpu.SemaphoreType.DMA,
        ],
    )(_gather_body)
    return run(tab, gidx)


def _fpath_body(f_ref, wm1_ref, bm1_ref, wsc_ref, bsc_ref,
                x1_ref, ysc_ref, ssc_ref, acc_ref):
    g = pl.program_id(0)
    fv = f_ref[...]
    x1 = jnp.dot(fv, wm1_ref[...], preferred_element_type=f32) + bm1_ref[...]
    x1_ref[...] = jnp.where(x1 >= 0, x1, 0.2 * x1)
    ysc = jnp.dot(fv, wsc_ref[...], preferred_element_type=f32) + bsc_ref[...]
    ysc_ref[...] = ysc

    @pl.when(g == 0)
    def _():
        acc_ref[...] = jnp.zeros_like(acc_ref)

    s = jnp.sum(ysc, axis=0, keepdims=True)
    ss = jnp.sum(ysc * ysc, axis=0, keepdims=True)
    acc_ref[...] += jnp.concatenate([s, ss], axis=0)

    @pl.when(g == pl.num_programs(0) - 1)
    def _():
        ssc_ref[...] = acc_ref[...]


def _fpath(fmat, wm1t, bm1, wsct, bsc):
    return pl.pallas_call(
        _fpath_body,
        grid=(P // TP,),
        in_specs=[
            pl.BlockSpec((TP, DIN), lambda g: (g, 0)),
            pl.BlockSpec((DIN, DIN), lambda g: (0, 0)),
            pl.BlockSpec((1, DIN), lambda g: (0, 0)),
            pl.BlockSpec((DIN, 2 * DOUT), lambda g: (0, 0)),
            pl.BlockSpec((1, 2 * DOUT), lambda g: (0, 0)),
        ],
        out_specs=[
            pl.BlockSpec((TP, DIN), lambda g: (g, 0)),
            pl.BlockSpec((TP, 2 * DOUT), lambda g: (g, 0)),
            pl.BlockSpec((2, 2 * DOUT), lambda g: (0, 0)),
        ],
        out_shape=[
            jax.ShapeDtypeStruct((P, DIN), f32),
            jax.ShapeDtypeStruct((P, 2 * DOUT), f32),
            jax.ShapeDtypeStruct((2, 2 * DOUT), f32),
        ],
        scratch_shapes=[pltpu.VMEM((2, 2 * DOUT), f32)],
    )(fmat, wm1t, bm1, wsct, bsc)


def _y_terms(c, nbr16, dist, wxt, wnt, wd, bv):
    cw = jnp.dot(c, wxt, preferred_element_type=f32)
    nb = nbr16[:, 0:3]
    nw_ = jnp.dot(nb, wnt, preferred_element_type=f32)
    y = (cw[:, None, :] + nw_.reshape(TP, K, D2)
         + dist[:, :, None] * wd.reshape(1, 1, D2) + bv.reshape(1, 1, D2))
    return y


def _encstats_body(c_ref, nbr_ref, dist_ref,
                   wxt1_ref, wnt1_ref, wd1_ref, b1_ref,
                   wxt2_ref, wnt2_ref, wd2_ref, b2_ref,
                   s1_ref, s2_ref, acc1_ref, acc2_ref):
    g = pl.program_id(0)
    c = c_ref[...]
    nbr16 = nbr_ref[...]
    dist = dist_ref[...]

    @pl.when(g == 0)
    def _():
        acc1_ref[...] = jnp.zeros_like(acc1_ref)
        acc2_ref[...] = jnp.zeros_like(acc2_ref)

    for (wxt, wnt, wd, bv, acc) in (
            (wxt1_ref, wnt1_ref, wd1_ref, b1_ref, acc1_ref),
            (wxt2_ref, wnt2_ref, wd2_ref, b2_ref, acc2_ref)):
        y = _y_terms(c, nbr16, dist, wxt[...], wnt[...], wd[...], bv[...])
        yf = y.reshape(TP * K, D2)
        s = jnp.sum(yf, axis=0, keepdims=True)
        ss = jnp.sum(yf * yf, axis=0, keepdims=True)
        acc[...] += jnp.concatenate([s, ss], axis=0)

    @pl.when(g == pl.num_programs(0) - 1)
    def _():
        s1_ref[...] = acc1_ref[...]
        s2_ref[...] = acc2_ref[...]


def _encstats(c2, nbr, dist, wxt1, wnt1, wd1, b1, wxt2, wnt2, wd2, b2):
    wspec = [
        pl.BlockSpec((3, D2), lambda g: (0, 0)),
        pl.BlockSpec((3, D2), lambda g: (0, 0)),
        pl.BlockSpec((1, D2), lambda g: (0, 0)),
        pl.BlockSpec((1, D2), lambda g: (0, 0)),
    ]
    return pl.pallas_call(
        _encstats_body,
        grid=(P // TP,),
        in_specs=[
            pl.BlockSpec((TP, 3), lambda g: (g, 0)),
            pl.BlockSpec((TP * K, 128), lambda g: (g, 0)),
            pl.BlockSpec((TP, K), lambda g: (g, 0)),
        ] + wspec + wspec,
        out_specs=[
            pl.BlockSpec((2, D2), lambda g: (0, 0)),
            pl.BlockSpec((2, D2), lambda g: (0, 0)),
        ],
        out_shape=[
            jax.ShapeDtypeStruct((2, D2), f32),
            jax.ShapeDtypeStruct((2, D2), f32),
        ],
        scratch_shapes=[pltpu.VMEM((2, D2), f32), pltpu.VMEM((2, D2), f32)],
    )(c2, nbr, dist, wxt1, wnt1, wd1, b1, wxt2, wnt2, wd2, b2)


def _bn_coeffs(stats, gv, bev, cnt):
    m = stats[0:1, :] / cnt
    v = stats[1:2, :] / cnt - m * m
    scale = gv / jnp.sqrt(v + EPS)
    shift = bev - m * scale
    return scale, shift


def _attpool(xb, wst, K_, TP_, C):
    s = jnp.dot(xb.reshape(TP_ * K_, C), wst,
                preferred_element_type=f32).reshape(TP_, K_, C)
    mx = jnp.max(s, axis=1, keepdims=True)
    e = jnp.exp(s - mx)
    rden = 1.0 / jnp.sum(e, axis=1, keepdims=True)
    return jnp.sum((e * rden) * xb, axis=1)


def _stage1_body(c_ref, nbr_ref, dist_ref, x1_ref, st1_ref,
                 wxt_ref, wnt_ref, wd_ref, b_ref, g1_ref, be1_ref,
                 wst_ref, wpt_ref, bp_ref,
                 z1_ref, sz_ref, acc_ref):
    g = pl.program_id(0)
    scale, shift = _bn_coeffs(st1_ref[...], g1_ref[...], be1_ref[...],
                              float(PK))
    y = _y_terms(c_ref[...], nbr_ref[...], dist_ref[...],
                 wxt_ref[...], wnt_ref[...], wd_ref[...], b_ref[...])
    enc = y * scale.reshape(1, 1, D2) + shift.reshape(1, 1, D2)
    enc = jnp.maximum(enc, 0.0)
    x1b = jnp.broadcast_to(x1_ref[...][:, None, :], (TP, K, D2))
    xb = jnp.concatenate([enc, x1b], axis=2)
    pooled = _attpool(xb, wst_ref[...], K, TP, DOUT)
    z = jnp.dot(pooled, wpt_ref[...], preferred_element_type=f32) + bp_ref[...]
    z1_ref[...] = z

    @pl.when(g == 0)
    def _():
        acc_ref[...] = jnp.zeros_like(acc_ref)

    s = jnp.sum(z, axis=0, keepdims=True)
    ss = jnp.sum(z * z, axis=0, keepdims=True)
    acc_ref[...] += jnp.concatenate([s, ss], axis=0)

    @pl.when(g == pl.num_programs(0) - 1)
    def _():
        sz_ref[...] = acc_ref[...]


def _stage1(c2, nbr, dist, x1, st1, wxt, wnt, wd, bv, g1, be1, wst, wpt, bp):
    return pl.pallas_call(
        _stage1_body,
        grid=(P // TP,),
        in_specs=[
            pl.BlockSpec((TP, 3), lambda g: (g, 0)),
            pl.BlockSpec((TP * K, 128), lambda g: (g, 0)),
            pl.BlockSpec((TP, K), lambda g: (g, 0)),
            pl.BlockSpec((TP, D2), lambda g: (g, 0)),
            pl.BlockSpec((2, D2), lambda g: (0, 0)),
            pl.BlockSpec((3, D2), lambda g: (0, 0)),
            pl.BlockSpec((3, D2), lambda g: (0, 0)),
            pl.BlockSpec((1, D2), lambda g: (0, 0)),
            pl.BlockSpec((1, D2), lambda g: (0, 0)),
            pl.BlockSpec((1, D2), lambda g: (0, 0)),
            pl.BlockSpec((1, D2), lambda g: (0, 0)),
            pl.BlockSpec((DOUT, DOUT), lambda g: (0, 0)),
            pl.BlockSpec((DOUT, D2), lambda g: (0, 0)),
            pl.BlockSpec((1, D2), lambda g: (0, 0)),
        ],
        out_specs=[
            pl.BlockSpec((TP, D2), lambda g: (g, 0)),
            pl.BlockSpec((2, D2), lambda g: (0, 0)),
        ],
        out_shape=[
            jax.ShapeDtypeStruct((P, D2), f32),
            jax.ShapeDtypeStruct((2, D2), f32),
        ],
        scratch_shapes=[pltpu.VMEM((2, D2), f32)],
    )(c2, nbr, dist, x1, st1, wxt, wnt, wd, bv, g1, be1, wst, wpt, bp)


def _stage2_body(c_ref, nbr_ref, dist_ref, z1_ref, sz1_ref, st2_ref,
                 gp1_ref, bep1_ref,
                 wxt_ref, wnt_ref, wd_ref, b_ref, g2_ref, be2_ref,
                 wst_ref, wpt_ref, bp_ref,
                 z2_ref, sz2_ref, acc_ref):
    g = pl.program_id(0)
    zscale, zshift = _bn_coeffs(sz1_ref[...], gp1_ref[...], bep1_ref[...],
                                float(P))
    x2 = jnp.maximum(z1_ref[...] * zscale + zshift, 0.0)
    escale, eshift = _bn_coeffs(st2_ref[...], g2_ref[...], be2_ref[...],
                                float(PK))
    y = _y_terms(c_ref[...], nbr_ref[...], dist_ref[...],
                 wxt_ref[...], wnt_ref[...], wd_ref[...], b_ref[...])
    enc = jnp.maximum(y * escale.reshape(1, 1, D2)
                      + eshift.reshape(1, 1, D2), 0.0)
    x2b = jnp.broadcast_to(x2[:, None, :], (TP, K, D2))
    xb = jnp.concatenate([enc, x2b], axis=2)
    pooled = _attpool(xb, wst_ref[...], K, TP, DOUT)
    z = jnp.dot(pooled, wpt_ref[...], preferred_element_type=f32) + bp_ref[...]
    z2_ref[...] = z

    @pl.when(g == 0)
    def _():
        acc_ref[...] = jnp.zeros_like(acc_ref)

    s = jnp.sum(z, axis=0, keepdims=True)
    ss = jnp.sum(z * z, axis=0, keepdims=True)
    acc_ref[...] += jnp.concatenate([s, ss], axis=0)

    @pl.when(g == pl.num_programs(0) - 1)
    def _():
        sz2_ref[...] = acc_ref[...]


def _stage2(c2, nbr, dist, z1, sz1, st2, gp1, bep1,
            wxt, wnt, wd, bv, g2, be2, wst, wpt, bp):
    return pl.pallas_call(
        _stage2_body,
        grid=(P // TP,),
        in_specs=[
            pl.BlockSpec((TP, 3), lambda g: (g, 0)),
            pl.BlockSpec((TP * K, 128), lambda g: (g, 0)),
            pl.BlockSpec((TP, K), lambda g: (g, 0)),
            pl.BlockSpec((TP, D2), lambda g: (g, 0)),
            pl.BlockSpec((2, D2), lambda g: (0, 0)),
            pl.BlockSpec((2, D2), lambda g: (0, 0)),
            pl.BlockSpec((1, D2), lambda g: (0, 0)),
            pl.BlockSpec((1, D2), lambda g: (0, 0)),
            pl.BlockSpec((3, D2), lambda g: (0, 0)),
            pl.BlockSpec((3, D2), lambda g: (0, 0)),
            pl.BlockSpec((1, D2), lambda g: (0, 0)),
            pl.BlockSpec((1, D2), lambda g: (0, 0)),
            pl.BlockSpec((1, D2), lambda g: (0, 0)),
            pl.BlockSpec((1, D2), lambda g: (0, 0)),
            pl.BlockSpec((DOUT, DOUT), lambda g: (0, 0)),
            pl.BlockSpec((DOUT, DOUT), lambda g: (0, 0)),
            pl.BlockSpec((1, DOUT), lambda g: (0, 0)),
        ],
        out_specs=[
            pl.BlockSpec((TP, DOUT), lambda g: (g, 0)),
            pl.BlockSpec((2, DOUT), lambda g: (0, 0)),
        ],
        out_shape=[
            jax.ShapeDtypeStruct((P, DOUT), f32),
            jax.ShapeDtypeStruct((2, DOUT), f32),
        ],
        scratch_shapes=[pltpu.VMEM((2, DOUT), f32)],
    )(c2, nbr, dist, z1, sz1, st2, gp1, bep1,
      wxt, wnt, wd, bv, g2, be2, wst, wpt, bp)


def _final_body(z2_ref, sz2_ref, gp2_ref, bep2_ref,
                ysc_ref, ssc_ref, gsc_ref, besc_ref,
                wm2_ref, bm2_ref, out_ref):
    zscale, zshift = _bn_coeffs(sz2_ref[...], gp2_ref[...], bep2_ref[...],
                                float(P))
    x3 = jnp.maximum(z2_ref[...] * zscale + zshift, 0.0)
    sscale, sshift = _bn_coeffs(ssc_ref[...], gsc_ref[...], besc_ref[...],
                                float(P))
    sc = ysc_ref[...] * sscale + sshift
    out = jnp.dot(x3, wm2_ref[...], preferred_element_type=f32) \
        + bm2_ref[...] + sc
    out_ref[...] = jnp.where(out >= 0, out, 0.01 * out)


def _final(z2, sz2, gp2, bep2, ysc, ssc, gsc, besc, wm2t, bm2):
    return pl.pallas_call(
        _final_body,
        grid=(P // TP,),
        in_specs=[
            pl.BlockSpec((TP, DOUT), lambda g: (g, 0)),
            pl.BlockSpec((2, DOUT), lambda g: (0, 0)),
            pl.BlockSpec((1, DOUT), lambda g: (0, 0)),
            pl.BlockSpec((1, DOUT), lambda g: (0, 0)),
            pl.BlockSpec((TP, 2 * DOUT), lambda g: (g, 0)),
            pl.BlockSpec((2, 2 * DOUT), lambda g: (0, 0)),
            pl.BlockSpec((1, 2 * DOUT), lambda g: (0, 0)),
            pl.BlockSpec((1, 2 * DOUT), lambda g: (0, 0)),
            pl.BlockSpec((DOUT, 2 * DOUT), lambda g: (0, 0)),
            pl.BlockSpec((1, 2 * DOUT), lambda g: (0, 0)),
        ],
        out_specs=pl.BlockSpec((TP, 2 * DOUT), lambda g: (g, 0)),
        out_shape=jax.ShapeDtypeStruct((P, 2 * DOUT), f32),
    )(z2, sz2, gp2, bep2, ysc, ssc, gsc, besc, wm2t, bm2)


def _split_lse(w):
    wx = (w[:, 0:3] + w[:, 6:9]).T
    wn = (w[:, 3:6] - w[:, 6:9]).T
    wd = w[:, 9].reshape(1, D2)
    return wx, wn, wd


def kernel(coords, features, W_mlp1, b_mlp1, W_lse1, b_lse1, g_lse1, be_lse1,
           W_score1, W_pool1, b_pool1, g_pool1, be_pool1, W_lse2, b_lse2,
           g_lse2, be_lse2, W_score2, W_pool2, b_pool2, g_pool2, be_pool2,
           W_mlp2, b_mlp2, W_sc, b_sc, g_sc, be_sc):
    c2 = coords.reshape(P, 3)
    ct = coords.transpose(0, 2, 1)
    fmat = features.reshape(B, DIN, N).transpose(0, 2, 1).reshape(P, DIN)
    tab = jnp.pad(c2, ((0, 0), (0, 125)))

    idx, dist = _knn(c2, ct)
    nbr = _gather_rows(tab, idx.reshape(PK))

    x1, ysc, ssc = _fpath(fmat, W_mlp1.T, b_mlp1.reshape(1, DIN),
                          W_sc.T, b_sc.reshape(1, 2 * DOUT))

    wx1, wn1, wd1 = _split_lse(W_lse1)
    wx2, wn2, wd2 = _split_lse(W_lse2)
    b1 = b_lse1.reshape(1, D2)
    b2 = b_lse2.reshape(1, D2)

    st1, st2 = _encstats(c2, nbr, dist, wx1, wn1, wd1, b1, wx2, wn2, wd2, b2)

    z1, sz1 = _stage1(c2, nbr, dist, x1, st1, wx1, wn1, wd1, b1,
                      g_lse1.reshape(1, D2), be_lse1.reshape(1, D2),
                      W_score1.T, W_pool1.T, b_pool1.reshape(1, D2))

    z2, sz2 = _stage2(c2, nbr, dist, z1, sz1, st2,
                      g_pool1.reshape(1, D2), be_pool1.reshape(1, D2),
                      wx2, wn2, wd2, b2,
                      g_lse2.reshape(1, D2), be_lse2.reshape(1, D2),
                      W_score2.T, W_pool2.T, b_pool2.reshape(1, DOUT))

    out = _final(z2, sz2, g_pool2.reshape(1, DOUT), be_pool2.reshape(1, DOUT),
                 ysc, ssc, g_sc.reshape(1, 2 * DOUT),
                 be_sc.reshape(1, 2 * DOUT),
                 W_mlp2.T, b_mlp2.reshape(1, 2 * DOUT))

    return out.reshape(B, N, 2 * DOUT).transpose(0, 2, 1)[:, :, :, None]
